# Initial kernel scaffold; baseline (speedup 1.0000x reference)
import jax
import jax.numpy as jnp
from jax import lax
from jax.experimental import pallas as pl
from jax.experimental.pallas import tpu as pltpu
from jax.experimental.pallas import tpu_sc as plsc

VOCAB = 100000
EMBED_DIM = 64
BATCH = 4096
SEQ = 200

NC = 2
NS = 16
NW = NC * NS
LANES = 16

B_PER_W = BATCH // NW
GCHUNK = 40
NG = SEQ // GCHUNK
ADDS = SEQ * EMBED_DIM // LANES


def _emb_kernel(ids_hbm, tok_hbm, pos_hbm, out_hbm,
                pos_v, idx_v, rows_v, gsem):
    wid = lax.axis_index("s") * NC + lax.axis_index("c")
    base = wid * B_PER_W

    pltpu.sync_copy(pos_hbm.at[pl.ds(0, SEQ)], pos_v)

    def body(g, _):
        b = base + g
        pltpu.sync_copy(ids_hbm.at[b], idx_v)
        for j in range(NG):
            pltpu.async_copy(
                tok_hbm.at[idx_v.at[pl.ds(j * GCHUNK, GCHUNK)]],
                rows_v.at[pl.ds(j * GCHUNK, GCHUNK)],
                gsem,
            )
        for j in range(NG):
            pltpu.make_async_copy(
                tok_hbm.at[idx_v.at[pl.ds(j * GCHUNK, GCHUNK)]],
                rows_v.at[pl.ds(j * GCHUNK, GCHUNK)],
                gsem,
            ).wait()

        def add_body(i, _):
            r = i // (EMBED_DIM // LANES)
            c = (i % (EMBED_DIM // LANES)) * LANES
            rows_v[r, pl.ds(c, LANES)] = (
                rows_v[r, pl.ds(c, LANES)] + pos_v[r, pl.ds(c, LANES)]
            )
            return 0

        lax.fori_loop(0, ADDS, add_body, 0, unroll=8)
        pltpu.sync_copy(rows_v, out_hbm.at[b])
        return 0

    lax.fori_loop(0, B_PER_W, body, 0)


@jax.jit
def _run(input_ids, token_embedding, position_embedding):
    mesh = plsc.VectorSubcoreMesh(core_axis_name="c", subcore_axis_name="s")
    call = pl.kernel(
        _emb_kernel,
        out_type=jax.ShapeDtypeStruct((BATCH, SEQ, EMBED_DIM), jnp.float32),
        mesh=mesh,
        scratch_types=[
            pltpu.VMEM((SEQ, EMBED_DIM), jnp.float32),
            pltpu.VMEM((SEQ,), jnp.int32),
            pltpu.VMEM((SEQ, EMBED_DIM), jnp.float32),
            pltpu.SemaphoreType.DMA,
        ],
    )
    return call(input_ids, token_embedding, position_embedding)


def kernel(input_ids, token_embedding, position_embedding):
    return _run(input_ids.astype(jnp.int32), token_embedding,
                position_embedding)


# speedup vs baseline: 2.2764x; 2.2764x over previous
import jax
import jax.numpy as jnp
from jax import lax
from jax.experimental import pallas as pl
from jax.experimental.pallas import tpu as pltpu
from jax.experimental.pallas import tpu_sc as plsc

VOCAB = 100000
EMBED_DIM = 64
BATCH = 4096
SEQ = 200

NC = 2
NS = 16
NW = NC * NS
LANES = 16

B_PER_W = BATCH // NW
GCHUNK = 40
NG = SEQ // GCHUNK
ADDS = SEQ * EMBED_DIM // LANES


def _emb_kernel(ids_hbm, tok_hbm, pos_hbm, out_hbm,
                pos_v, idx_v, rows_v, gsem):
    wid = lax.axis_index("s") * NC + lax.axis_index("c")
    base = wid * B_PER_W

    pltpu.sync_copy(pos_hbm.at[pl.ds(0, SEQ)], pos_v)

    def body(g, _):
        b = base + g
        pltpu.sync_copy(ids_hbm.at[b], idx_v)
        for j in range(NG):
            pltpu.async_copy(
                tok_hbm.at[idx_v.at[pl.ds(j * GCHUNK, GCHUNK)]],
                rows_v.at[pl.ds(j * GCHUNK, GCHUNK)],
                gsem,
            )
        for j in range(NG):
            pltpu.make_async_copy(
                tok_hbm.at[idx_v.at[pl.ds(j * GCHUNK, GCHUNK)]],
                rows_v.at[pl.ds(j * GCHUNK, GCHUNK)],
                gsem,
            ).wait()

        def add_body(i, _):
            r = i // (EMBED_DIM // LANES)
            c = (i % (EMBED_DIM // LANES)) * LANES
            rows_v[r, pl.ds(c, LANES)] = (
                rows_v[r, pl.ds(c, LANES)] + pos_v[r, pl.ds(c, LANES)]
            )
            return 0

        lax.fori_loop(0, ADDS, add_body, 0, unroll=8)
        pltpu.sync_copy(rows_v, out_hbm.at[b])
        return 0

    lax.fori_loop(0, B_PER_W, body, 0)


@jax.jit
def _run(input_ids, token_embedding, position_embedding):
    mesh = plsc.VectorSubcoreMesh(core_axis_name="c", subcore_axis_name="s")
    call = pl.kernel(
        _emb_kernel,
        out_type=jax.ShapeDtypeStruct((BATCH, SEQ, EMBED_DIM), jnp.float32),
        mesh=mesh,
        scratch_types=[
            pltpu.VMEM((SEQ, EMBED_DIM), jnp.float32),
            pltpu.VMEM((SEQ,), jnp.int32),
            pltpu.VMEM((SEQ, EMBED_DIM), jnp.float32),
            pltpu.SemaphoreType.DMA,
        ],
        compiler_params=pltpu.CompilerParams(use_tc_tiling_on_sc=False),
    )
    return call(input_ids, token_embedding, position_embedding)


def kernel(input_ids, token_embedding, position_embedding):
    return _run(input_ids.astype(jnp.int32), token_embedding,
                position_embedding)


# speedup vs baseline: 2.7363x; 1.2021x over previous
import jax
import jax.numpy as jnp
from jax import lax
from jax.experimental import pallas as pl
from jax.experimental.pallas import tpu as pltpu
from jax.experimental.pallas import tpu_sc as plsc

VOCAB = 100000
EMBED_DIM = 64
BATCH = 4096
SEQ = 200

NC = 2
NS = 16
NW = NC * NS
LANES = 16
CPR = EMBED_DIM // LANES

ROWS_PER_W = BATCH // NW
SC_ROWS = 2
CHUNK = SC_ROWS * SEQ
GCHUNK = 80
NG = CHUNK // GCHUNK
N_ITEMS = ROWS_PER_W // SC_ROWS
IDX_PER_W = ROWS_PER_W * SEQ


def _emb_kernel(ids_hbm, tok_hbm, pos_hbm, out_hbm,
                pos_v, idx_v, rows_a, rows_b, gsem_a, gsem_b,
                osem_a, osem_b):
    wid = lax.axis_index("s") * NC + lax.axis_index("c")
    base = wid * IDX_PER_W

    pltpu.sync_copy(ids_hbm.at[pl.ds(base, IDX_PER_W)], idx_v)
    pltpu.sync_copy(pos_hbm.at[pl.ds(0, SEQ)], pos_v)

    def fire_gathers(item, rows_v, sem):
        off = item * CHUNK
        for j in range(NG):
            pltpu.async_copy(
                tok_hbm.at[idx_v.at[pl.ds(off + j * GCHUNK, GCHUNK)]],
                rows_v.at[pl.ds(j * GCHUNK, GCHUNK)],
                sem,
            )

    def wait_gathers(item, rows_v, sem):
        off = item * CHUNK
        for j in range(NG):
            pltpu.make_async_copy(
                tok_hbm.at[idx_v.at[pl.ds(off + j * GCHUNK, GCHUNK)]],
                rows_v.at[pl.ds(j * GCHUNK, GCHUNK)],
                sem,
            ).wait()

    def add_pos(rows_v):
        def row_body(r, _):
            for half in range(SC_ROWS):
                for c in range(CPR):
                    rows_v[half * SEQ + r, pl.ds(c * LANES, LANES)] = (
                        rows_v[half * SEQ + r, pl.ds(c * LANES, LANES)]
                        + pos_v[r, pl.ds(c * LANES, LANES)]
                    )
            return 0
        lax.fori_loop(0, SEQ, row_body, 0, unroll=2)

    def fire_store(item, rows_v, sem):
        pltpu.async_copy(
            rows_v, out_hbm.at[pl.ds(base + item * CHUNK, CHUNK)], sem)

    def wait_store(item, rows_v, sem):
        pltpu.make_async_copy(
            rows_v, out_hbm.at[pl.ds(base + item * CHUNK, CHUNK)], sem,
        ).wait()

    fire_gathers(0, rows_a, gsem_a)
    fire_gathers(1, rows_b, gsem_b)

    def body(g, _):
        ia = 2 * g
        ib = 2 * g + 1
        wait_gathers(ia, rows_a, gsem_a)
        add_pos(rows_a)
        fire_store(ia, rows_a, osem_a)
        wait_gathers(ib, rows_b, gsem_b)
        add_pos(rows_b)
        fire_store(ib, rows_b, osem_b)

        @pl.when(g < N_ITEMS // 2 - 1)
        def _refill():
            wait_store(ia, rows_a, osem_a)
            fire_gathers(ia + 2, rows_a, gsem_a)
            wait_store(ib, rows_b, osem_b)
            fire_gathers(ib + 2, rows_b, gsem_b)

        return 0

    lax.fori_loop(0, N_ITEMS // 2, body, 0)

    wait_store(N_ITEMS - 2, rows_a, osem_a)
    wait_store(N_ITEMS - 1, rows_b, osem_b)


@jax.jit
def _run(input_ids, token_embedding, position_embedding):
    mesh = plsc.VectorSubcoreMesh(core_axis_name="c", subcore_axis_name="s")
    call = pl.kernel(
        _emb_kernel,
        out_type=jax.ShapeDtypeStruct((BATCH * SEQ, EMBED_DIM), jnp.float32),
        mesh=mesh,
        scratch_types=[
            pltpu.VMEM((SEQ, EMBED_DIM), jnp.float32),
            pltpu.VMEM((IDX_PER_W,), jnp.int32),
            pltpu.VMEM((CHUNK, EMBED_DIM), jnp.float32),
            pltpu.VMEM((CHUNK, EMBED_DIM), jnp.float32),
            pltpu.SemaphoreType.DMA,
            pltpu.SemaphoreType.DMA,
            pltpu.SemaphoreType.DMA,
            pltpu.SemaphoreType.DMA,
        ],
        compiler_params=pltpu.CompilerParams(use_tc_tiling_on_sc=False),
    )
    out = call(input_ids.reshape(BATCH * SEQ), token_embedding,
               position_embedding)
    return out.reshape(BATCH, SEQ, EMBED_DIM)


def kernel(input_ids, token_embedding, position_embedding):
    return _run(input_ids.astype(jnp.int32), token_embedding,
                position_embedding)
